# baseline (speedup 1.0000x reference)
import functools

import jax
import jax.numpy as jnp
from jax import lax
from jax.experimental import pallas as pl
from jax.experimental.pallas import tpu as pltpu
from jax.experimental.pallas import tpu_sc as plsc

NUM_EXPERTS = 16
TOP_K = 2
HIDDEN = 2048
INTER = 768
TOKENS = 8192
ROWS = TOKENS * TOP_K

BM = 256
NUM_BLOCKS = ROWS // BM
MAX_TILES = NUM_BLOCKS + NUM_EXPERTS

_NEG = float(jnp.finfo(jnp.float32).min)


def _routing_body(logits_ref, w_ref, d_ref, bounds_ref):
    logits = logits_ref[...]
    lanes = jax.lax.broadcasted_iota(jnp.int32, (TOKENS, NUM_EXPERTS), 1)

    v1 = jnp.max(logits, axis=1, keepdims=True)
    e1 = jnp.min(jnp.where(logits == v1, lanes, NUM_EXPERTS), axis=1,
                 keepdims=True)
    oh1 = (lanes == e1)
    masked = jnp.where(oh1, _NEG, logits)
    v2 = jnp.max(masked, axis=1, keepdims=True)
    e2 = jnp.min(jnp.where(masked == v2, lanes, NUM_EXPERTS), axis=1,
                 keepdims=True)
    oh2 = (lanes == e2)

    w0 = jax.nn.sigmoid(v1 - v2)
    w1 = jax.nn.sigmoid(v2 - v1)

    s = (oh1 | oh2).astype(jnp.int32)
    c = jnp.concatenate(
        [jnp.zeros((1, NUM_EXPERTS), jnp.int32), s[:-1]], axis=0)
    sh = 1
    while sh < TOKENS:
        c = c + jnp.concatenate(
            [jnp.zeros((sh, NUM_EXPERTS), jnp.int32), c[:-sh]], axis=0)
        sh *= 2

    tot = c[-1:] + s[-1:]
    st = jnp.concatenate([jnp.zeros((1, 1), jnp.int32), tot[:, :-1]], axis=1)
    sh = 1
    while sh < NUM_EXPERTS:
        st = st + jnp.concatenate(
            [jnp.zeros((1, sh), jnp.int32), st[:, :-sh]], axis=1)
        sh *= 2

    pos = c + st
    d0 = jnp.sum(jnp.where(oh1, pos, 0), axis=1, keepdims=True)
    d1 = jnp.sum(jnp.where(oh2, pos, 0), axis=1, keepdims=True)

    w_ref[...] = jnp.concatenate([w0, w1], axis=1)
    d_ref[...] = jnp.concatenate([d0, d1], axis=1)
    bounds_ref[...] = jnp.concatenate([st, st + tot], axis=0)


def _routing(router_logits):
    return pl.pallas_call(
        _routing_body,
        out_shape=(
            jax.ShapeDtypeStruct((TOKENS, TOP_K), jnp.float32),
            jax.ShapeDtypeStruct((TOKENS, TOP_K), jnp.int32),
            jax.ShapeDtypeStruct((2, NUM_EXPERTS), jnp.int32),
        ),
    )(router_logits)


NC = 1


def _moe_mm_body(te_ref, tb_ref, bounds_ref, hs_ref, g_ref, u_ref, d_ref,
                 ws_ref, out_ref, acc_ref):
    t = pl.program_id(0)
    nc = pl.program_id(1)
    e = te_ref[t]
    b = tb_ref[t]
    start = bounds_ref[0, e]
    end = bounds_ref[1, e]
    rows = b * BM + jax.lax.broadcasted_iota(jnp.int32, (BM, 1), 0)
    mask = (rows >= start) & (rows < end)

    x = hs_ref[...].astype(jnp.bfloat16)
    g = jnp.dot(x, g_ref[0].astype(jnp.bfloat16),
                preferred_element_type=jnp.float32)
    u = jnp.dot(x, u_ref[0].astype(jnp.bfloat16),
                preferred_element_type=jnp.float32)
    act = (g * jax.nn.sigmoid(g)) * u
    act = act * ws_ref[0, 0].reshape(BM, 1)
    o = jnp.dot(act.astype(jnp.bfloat16), d_ref[0].astype(jnp.bfloat16),
                preferred_element_type=jnp.float32)

    if NC > 1:
        @pl.when(nc == 0)
        def _():
            acc_ref[...] = o

    @pl.when(nc == NC - 1)
    def _():
        prev = o if NC == 1 else acc_ref[...] + o
        out_ref[...] = jnp.where(mask, prev, out_ref[...])


def _grouped_mlp(hs_sorted, gate_proj, up_proj, down_proj, w_sorted, bounds):
    starts = bounds[0]
    ends = bounds[1]
    group_sizes = ends - starts
    first = starts // BM
    last = jnp.maximum(ends - 1, 0) // BM
    nblk = jnp.where(group_sizes > 0, last - first + 1, 0)
    cum = jnp.cumsum(nblk)
    cum0 = cum - nblk
    t = jnp.arange(MAX_TILES, dtype=jnp.int32)
    tile_expert = jnp.searchsorted(cum, t, side="right").astype(jnp.int32)
    tile_expert = jnp.clip(tile_expert, 0, NUM_EXPERTS - 1)
    tile_block = first[tile_expert] + (t - cum0[tile_expert])
    tile_block = jnp.clip(tile_block, 0, NUM_BLOCKS - 1).astype(jnp.int32)

    grid_spec = pltpu.PrefetchScalarGridSpec(
        num_scalar_prefetch=3,
        grid=(MAX_TILES, NC),
        in_specs=[
            pl.BlockSpec((BM, HIDDEN),
                         lambda t, nc, te, tb, bd: (tb[t], 0)),
            pl.BlockSpec((1, HIDDEN, INTER // NC),
                         lambda t, nc, te, tb, bd: (te[t], 0, nc)),
            pl.BlockSpec((1, HIDDEN, INTER // NC),
                         lambda t, nc, te, tb, bd: (te[t], 0, nc)),
            pl.BlockSpec((1, INTER // NC, HIDDEN),
                         lambda t, nc, te, tb, bd: (te[t], nc, 0)),
            pl.BlockSpec((1, 1, BM),
                         lambda t, nc, te, tb, bd: (tb[t], 0, 0)),
        ],
        out_specs=pl.BlockSpec((BM, HIDDEN),
                               lambda t, nc, te, tb, bd: (tb[t], 0)),
        scratch_shapes=[pltpu.VMEM((BM, HIDDEN), jnp.float32)],
    )
    return pl.pallas_call(
        _moe_mm_body,
        grid_spec=grid_spec,
        out_shape=jax.ShapeDtypeStruct((ROWS, HIDDEN), jnp.float32),
        compiler_params=pltpu.CompilerParams(
            dimension_semantics=("arbitrary", "arbitrary"),
            vmem_limit_bytes=120 * 1024 * 1024),
    )(tile_expert, tile_block, bounds, hs_sorted, gate_proj, up_proj,
      down_proj, w_sorted.reshape(NUM_BLOCKS, 1, BM))


_SC_WORKERS = 32
_TPW = TOKENS // _SC_WORKERS
_CH = 8
_NCH = _TPW // _CH
_IDX_ROWS = TOKENS // _CH


def _combine_sc(down_out, d0, d1):
    mesh = plsc.VectorSubcoreMesh(core_axis_name="c", subcore_axis_name="s")

    @functools.partial(
        pl.kernel, mesh=mesh,
        out_type=jax.ShapeDtypeStruct((TOKENS, HIDDEN), jnp.float32),
        scratch_types=[
            pltpu.VMEM((_NCH, _CH), jnp.int32),
            pltpu.VMEM((_NCH, _CH), jnp.int32),
            pltpu.VMEM((_CH, HIDDEN), jnp.float32),
            pltpu.VMEM((_CH, HIDDEN), jnp.float32),
            pltpu.VMEM((_CH, HIDDEN), jnp.float32),
            pltpu.VMEM((_CH, HIDDEN), jnp.float32),
            pltpu.SemaphoreType.DMA,
            pltpu.SemaphoreType.DMA,
            pltpu.SemaphoreType.DMA,
            pltpu.SemaphoreType.DMA,
            pltpu.SemaphoreType.DMA,
            pltpu.SemaphoreType.DMA,
        ],
    )
    def k(x_hbm, d0_hbm, d1_hbm, out_hbm, idx0_v, idx1_v,
          a0, a1, b0, b1, ga0, ga1, gb0, gb1, sa, sb):
        wid = lax.axis_index("s") * 2 + lax.axis_index("c")
        row0 = wid * _NCH
        tok0 = wid * _TPW
        pltpu.sync_copy(d0_hbm.at[pl.ds(row0, _NCH)], idx0_v)
        pltpu.sync_copy(d1_hbm.at[pl.ds(row0, _NCH)], idx1_v)

        def gather(c, bufs, sems):
            pltpu.async_copy(x_hbm.at[idx0_v.at[c]], bufs[0], sems[0])
            pltpu.async_copy(x_hbm.at[idx1_v.at[c]], bufs[1], sems[1])

        def wait_gather(bufs, sems):
            pltpu.make_async_copy(x_hbm.at[idx0_v.at[0]], bufs[0],
                                  sems[0]).wait()
            pltpu.make_async_copy(x_hbm.at[idx1_v.at[0]], bufs[1],
                                  sems[1]).wait()

        def add_rows(dst, src):
            for r in range(_CH):
                def inner(i, _):
                    base = i * 128
                    for kk in range(8):
                        off = base + kk * 16
                        dst[r, pl.ds(off, 16)] = (dst[r, pl.ds(off, 16)] +
                                                  src[r, pl.ds(off, 16)])
                    return 0
                lax.fori_loop(0, HIDDEN // 128, inner, 0)

        def store(c, buf, sem):
            pltpu.async_copy(buf, out_hbm.at[pl.ds(tok0 + c * _CH, _CH)], sem)

        def wait_store(c, buf, sem):
            pltpu.make_async_copy(buf, out_hbm.at[pl.ds(tok0 + c * _CH, _CH)],
                                  sem).wait()

        gather(0, (a0, a1), (ga0, ga1))
        gather(1, (b0, b1), (gb0, gb1))

        def body(i, _):
            ca = 2 * i
            cb = 2 * i + 1
            wait_gather((a0, a1), (ga0, ga1))
            add_rows(a0, a1)
            store(ca, a0, sa)

            @pl.when(i < _NCH // 2 - 1)
            def _():
                wait_store(ca, a0, sa)
                gather(ca + 2, (a0, a1), (ga0, ga1))

            wait_gather((b0, b1), (gb0, gb1))
            add_rows(b0, b1)
            store(cb, b0, sb)

            @pl.when(i < _NCH // 2 - 1)
            def _():
                wait_store(cb, b0, sb)
                gather(cb + 2, (b0, b1), (gb0, gb1))

            return 0

        lax.fori_loop(0, _NCH // 2, body, 0)
        wait_store(_NCH - 2, a0, sa)
        wait_store(_NCH - 1, b0, sb)

    return k(down_out, d0.reshape(_IDX_ROWS, _CH), d1.reshape(_IDX_ROWS, _CH))


@jax.jit
def kernel(hidden_states, router_logits, gate_proj, up_proj, down_proj):
    w, d, bounds = _routing(router_logits)

    d_flat = d.ravel()
    src = jnp.zeros((ROWS,), jnp.int32).at[d_flat].set(
        jnp.arange(ROWS, dtype=jnp.int32))
    hs_sorted = hidden_states[src // TOP_K]
    w_sorted = w.ravel()[src]

    down_out = _grouped_mlp(hs_sorted, gate_proj, up_proj, down_proj,
                            w_sorted, bounds)

    return _combine_sc(down_out, d[:, 0] + 0, d[:, 1] + 0)


# speedup vs baseline: 1.1481x; 1.1481x over previous
import functools

import jax
import jax.numpy as jnp
from jax import lax
from jax.experimental import pallas as pl
from jax.experimental.pallas import tpu as pltpu
from jax.experimental.pallas import tpu_sc as plsc

NUM_EXPERTS = 16
TOP_K = 2
HIDDEN = 2048
INTER = 768
TOKENS = 8192
ROWS = TOKENS * TOP_K

BM = 512
NUM_BLOCKS = ROWS // BM
MAX_TILES = NUM_BLOCKS + NUM_EXPERTS

_NEG = float(jnp.finfo(jnp.float32).min)


def _routing_body(logits_ref, w_ref, d_ref, bounds_ref):
    logits = logits_ref[...]
    lanes = jax.lax.broadcasted_iota(jnp.int32, (TOKENS, NUM_EXPERTS), 1)

    v1 = jnp.max(logits, axis=1, keepdims=True)
    e1 = jnp.min(jnp.where(logits == v1, lanes, NUM_EXPERTS), axis=1,
                 keepdims=True)
    oh1 = (lanes == e1)
    masked = jnp.where(oh1, _NEG, logits)
    v2 = jnp.max(masked, axis=1, keepdims=True)
    e2 = jnp.min(jnp.where(masked == v2, lanes, NUM_EXPERTS), axis=1,
                 keepdims=True)
    oh2 = (lanes == e2)

    w0 = jax.nn.sigmoid(v1 - v2)
    w1 = jax.nn.sigmoid(v2 - v1)

    s = (oh1 | oh2).astype(jnp.int32)
    c = jnp.concatenate(
        [jnp.zeros((1, NUM_EXPERTS), jnp.int32), s[:-1]], axis=0)
    sh = 1
    while sh < TOKENS:
        c = c + jnp.concatenate(
            [jnp.zeros((sh, NUM_EXPERTS), jnp.int32), c[:-sh]], axis=0)
        sh *= 2

    tot = c[-1:] + s[-1:]
    st = jnp.concatenate([jnp.zeros((1, 1), jnp.int32), tot[:, :-1]], axis=1)
    sh = 1
    while sh < NUM_EXPERTS:
        st = st + jnp.concatenate(
            [jnp.zeros((1, sh), jnp.int32), st[:, :-sh]], axis=1)
        sh *= 2

    pos = c + st
    d0 = jnp.sum(jnp.where(oh1, pos, 0), axis=1, keepdims=True)
    d1 = jnp.sum(jnp.where(oh2, pos, 0), axis=1, keepdims=True)

    w_ref[...] = jnp.concatenate([w0, w1], axis=1)
    d_ref[...] = jnp.concatenate([d0, d1], axis=1)
    bounds_ref[...] = jnp.concatenate([st, st + tot], axis=0)


def _routing(router_logits):
    return pl.pallas_call(
        _routing_body,
        out_shape=(
            jax.ShapeDtypeStruct((TOKENS, TOP_K), jnp.float32),
            jax.ShapeDtypeStruct((TOKENS, TOP_K), jnp.int32),
            jax.ShapeDtypeStruct((2, NUM_EXPERTS), jnp.int32),
        ),
    )(router_logits)


NC = 1


def _moe_mm_body(te_ref, tb_ref, bounds_ref, hs_ref, g_ref, u_ref, d_ref,
                 ws_ref, out_ref, acc_ref):
    t = pl.program_id(0)
    nc = pl.program_id(1)
    e = te_ref[t]
    b = tb_ref[t]
    start = bounds_ref[0, e]
    end = bounds_ref[1, e]
    rows = b * BM + jax.lax.broadcasted_iota(jnp.int32, (BM, 1), 0)
    mask = (rows >= start) & (rows < end)

    x = hs_ref[...].astype(jnp.bfloat16)
    g = jnp.dot(x, g_ref[0].astype(jnp.bfloat16),
                preferred_element_type=jnp.float32)
    u = jnp.dot(x, u_ref[0].astype(jnp.bfloat16),
                preferred_element_type=jnp.float32)
    act = (g * jax.nn.sigmoid(g)) * u
    act = act * ws_ref[0, 0].reshape(BM, 1)
    o = jnp.dot(act.astype(jnp.bfloat16), d_ref[0].astype(jnp.bfloat16),
                preferred_element_type=jnp.float32)

    if NC > 1:
        @pl.when(nc == 0)
        def _():
            acc_ref[...] = o

    @pl.when(nc == NC - 1)
    def _():
        prev = o if NC == 1 else acc_ref[...] + o
        out_ref[...] = jnp.where(mask, prev, out_ref[...])


def _grouped_mlp(hs_sorted, gate_proj, up_proj, down_proj, w_sorted, bounds):
    starts = bounds[0]
    ends = bounds[1]
    group_sizes = ends - starts
    first = starts // BM
    last = jnp.maximum(ends - 1, 0) // BM
    nblk = jnp.where(group_sizes > 0, last - first + 1, 0)
    cum = jnp.cumsum(nblk)
    cum0 = cum - nblk
    t = jnp.arange(MAX_TILES, dtype=jnp.int32)
    tile_expert = jnp.searchsorted(cum, t, side="right").astype(jnp.int32)
    tile_expert = jnp.clip(tile_expert, 0, NUM_EXPERTS - 1)
    tile_block = first[tile_expert] + (t - cum0[tile_expert])
    tile_block = jnp.clip(tile_block, 0, NUM_BLOCKS - 1).astype(jnp.int32)

    grid_spec = pltpu.PrefetchScalarGridSpec(
        num_scalar_prefetch=3,
        grid=(MAX_TILES, NC),
        in_specs=[
            pl.BlockSpec((BM, HIDDEN),
                         lambda t, nc, te, tb, bd: (tb[t], 0)),
            pl.BlockSpec((1, HIDDEN, INTER // NC),
                         lambda t, nc, te, tb, bd: (te[t], 0, nc)),
            pl.BlockSpec((1, HIDDEN, INTER // NC),
                         lambda t, nc, te, tb, bd: (te[t], 0, nc)),
            pl.BlockSpec((1, INTER // NC, HIDDEN),
                         lambda t, nc, te, tb, bd: (te[t], nc, 0)),
            pl.BlockSpec((1, 1, BM),
                         lambda t, nc, te, tb, bd: (tb[t], 0, 0)),
        ],
        out_specs=pl.BlockSpec((BM, HIDDEN),
                               lambda t, nc, te, tb, bd: (tb[t], 0)),
        scratch_shapes=[pltpu.VMEM((BM, HIDDEN), jnp.float32)],
    )
    return pl.pallas_call(
        _moe_mm_body,
        grid_spec=grid_spec,
        out_shape=jax.ShapeDtypeStruct((ROWS, HIDDEN), jnp.float32),
        compiler_params=pltpu.CompilerParams(
            dimension_semantics=("arbitrary", "arbitrary"),
            vmem_limit_bytes=120 * 1024 * 1024),
    )(tile_expert, tile_block, bounds, hs_sorted, gate_proj, up_proj,
      down_proj, w_sorted.reshape(NUM_BLOCKS, 1, BM))


_SC_WORKERS = 32
_TPW = TOKENS // _SC_WORKERS
_CH = 8
_NCH = _TPW // _CH
_IDX_ROWS = TOKENS // _CH


def _scatter_sc(hidden_states, d0, d1, w0, w1):
    mesh = plsc.VectorSubcoreMesh(core_axis_name="c", subcore_axis_name="s")

    @functools.partial(
        pl.kernel, mesh=mesh,
        out_type=(
            jax.ShapeDtypeStruct((ROWS, HIDDEN), jnp.float32),
            jax.ShapeDtypeStruct((ROWS,), jnp.float32),
        ),
        scratch_types=[
            pltpu.VMEM((_NCH, _CH), jnp.int32),
            pltpu.VMEM((_NCH, _CH), jnp.int32),
            pltpu.VMEM((_NCH, _CH), jnp.float32),
            pltpu.VMEM((_NCH, _CH), jnp.float32),
            pltpu.VMEM((_CH, HIDDEN), jnp.float32),
            pltpu.VMEM((_CH, HIDDEN), jnp.float32),
            pltpu.SemaphoreType.DMA,
            pltpu.SemaphoreType.DMA,
            pltpu.SemaphoreType.DMA,
            pltpu.SemaphoreType.DMA,
            pltpu.SemaphoreType.DMA,
            pltpu.SemaphoreType.DMA,
            pltpu.SemaphoreType.DMA,
            pltpu.SemaphoreType.DMA,
            pltpu.SemaphoreType.DMA,
            pltpu.SemaphoreType.DMA,
        ],
    )
    def k(hs_hbm, d0_hbm, d1_hbm, w0_hbm, w1_hbm, hss_hbm, ws_hbm,
          idx0_v, idx1_v, w0_v, w1_v, bufa, bufb,
          la, lb, sa0, sa1, sb0, sb1, wa0, wa1, wb0, wb1):
        wid = lax.axis_index("s") * 2 + lax.axis_index("c")
        row0 = wid * _NCH
        tok0 = wid * _TPW
        pltpu.sync_copy(d0_hbm.at[pl.ds(row0, _NCH)], idx0_v)
        pltpu.sync_copy(d1_hbm.at[pl.ds(row0, _NCH)], idx1_v)
        pltpu.sync_copy(w0_hbm.at[pl.ds(row0, _NCH)], w0_v)
        pltpu.sync_copy(w1_hbm.at[pl.ds(row0, _NCH)], w1_v)

        def load(c, buf, sem):
            pltpu.async_copy(hs_hbm.at[pl.ds(tok0 + c * _CH, _CH)], buf, sem)

        def wait_load(buf, sem):
            pltpu.make_async_copy(hs_hbm.at[pl.ds(0, _CH)], buf, sem).wait()

        def scatter(c, buf, sems):
            pltpu.async_copy(buf, hss_hbm.at[idx0_v.at[c]], sems[0])
            pltpu.async_copy(buf, hss_hbm.at[idx1_v.at[c]], sems[1])
            pltpu.async_copy(w0_v.at[c], ws_hbm.at[idx0_v.at[c]], sems[2])
            pltpu.async_copy(w1_v.at[c], ws_hbm.at[idx1_v.at[c]], sems[3])

        def wait_scatter(buf, sems):
            pltpu.make_async_copy(buf, hss_hbm.at[idx0_v.at[0]],
                                  sems[0]).wait()
            pltpu.make_async_copy(buf, hss_hbm.at[idx1_v.at[0]],
                                  sems[1]).wait()
            pltpu.make_async_copy(w0_v.at[0], ws_hbm.at[idx0_v.at[0]],
                                  sems[2]).wait()
            pltpu.make_async_copy(w1_v.at[0], ws_hbm.at[idx1_v.at[0]],
                                  sems[3]).wait()

        load(0, bufa, la)
        load(1, bufb, lb)

        def body(i, _):
            ca = 2 * i
            cb = 2 * i + 1
            wait_load(bufa, la)
            scatter(ca, bufa, (sa0, sa1, wa0, wa1))

            @pl.when(i < _NCH // 2 - 1)
            def _():
                wait_scatter(bufa, (sa0, sa1, wa0, wa1))
                load(ca + 2, bufa, la)

            wait_load(bufb, lb)
            scatter(cb, bufb, (sb0, sb1, wb0, wb1))

            @pl.when(i < _NCH // 2 - 1)
            def _():
                wait_scatter(bufb, (sb0, sb1, wb0, wb1))
                load(cb + 2, bufb, lb)

            return 0

        lax.fori_loop(0, _NCH // 2, body, 0)
        wait_scatter(bufa, (sa0, sa1, wa0, wa1))
        wait_scatter(bufb, (sb0, sb1, wb0, wb1))

    return k(hidden_states,
             d0.reshape(_IDX_ROWS, _CH), d1.reshape(_IDX_ROWS, _CH),
             w0.reshape(_IDX_ROWS, _CH), w1.reshape(_IDX_ROWS, _CH))


def _combine_sc(down_out, d0, d1):
    mesh = plsc.VectorSubcoreMesh(core_axis_name="c", subcore_axis_name="s")

    @functools.partial(
        pl.kernel, mesh=mesh,
        out_type=jax.ShapeDtypeStruct((TOKENS, HIDDEN), jnp.float32),
        scratch_types=[
            pltpu.VMEM((_NCH, _CH), jnp.int32),
            pltpu.VMEM((_NCH, _CH), jnp.int32),
            pltpu.VMEM((_CH, HIDDEN), jnp.float32),
            pltpu.VMEM((_CH, HIDDEN), jnp.float32),
            pltpu.VMEM((_CH, HIDDEN), jnp.float32),
            pltpu.VMEM((_CH, HIDDEN), jnp.float32),
            pltpu.SemaphoreType.DMA,
            pltpu.SemaphoreType.DMA,
            pltpu.SemaphoreType.DMA,
            pltpu.SemaphoreType.DMA,
            pltpu.SemaphoreType.DMA,
            pltpu.SemaphoreType.DMA,
        ],
    )
    def k(x_hbm, d0_hbm, d1_hbm, out_hbm, idx0_v, idx1_v,
          a0, a1, b0, b1, ga0, ga1, gb0, gb1, sa, sb):
        wid = lax.axis_index("s") * 2 + lax.axis_index("c")
        row0 = wid * _NCH
        tok0 = wid * _TPW
        pltpu.sync_copy(d0_hbm.at[pl.ds(row0, _NCH)], idx0_v)
        pltpu.sync_copy(d1_hbm.at[pl.ds(row0, _NCH)], idx1_v)

        def gather(c, bufs, sems):
            pltpu.async_copy(x_hbm.at[idx0_v.at[c]], bufs[0], sems[0])
            pltpu.async_copy(x_hbm.at[idx1_v.at[c]], bufs[1], sems[1])

        def wait_gather(bufs, sems):
            pltpu.make_async_copy(x_hbm.at[idx0_v.at[0]], bufs[0],
                                  sems[0]).wait()
            pltpu.make_async_copy(x_hbm.at[idx1_v.at[0]], bufs[1],
                                  sems[1]).wait()

        def add_rows(dst, src):
            for r in range(_CH):
                def inner(i, _):
                    base = i * 128
                    for kk in range(8):
                        off = base + kk * 16
                        dst[r, pl.ds(off, 16)] = (dst[r, pl.ds(off, 16)] +
                                                  src[r, pl.ds(off, 16)])
                    return 0
                lax.fori_loop(0, HIDDEN // 128, inner, 0)

        def store(c, buf, sem):
            pltpu.async_copy(buf, out_hbm.at[pl.ds(tok0 + c * _CH, _CH)], sem)

        def wait_store(c, buf, sem):
            pltpu.make_async_copy(buf, out_hbm.at[pl.ds(tok0 + c * _CH, _CH)],
                                  sem).wait()

        gather(0, (a0, a1), (ga0, ga1))
        gather(1, (b0, b1), (gb0, gb1))

        def body(i, _):
            ca = 2 * i
            cb = 2 * i + 1
            wait_gather((a0, a1), (ga0, ga1))
            add_rows(a0, a1)
            store(ca, a0, sa)

            @pl.when(i < _NCH // 2 - 1)
            def _():
                wait_store(ca, a0, sa)
                gather(ca + 2, (a0, a1), (ga0, ga1))

            wait_gather((b0, b1), (gb0, gb1))
            add_rows(b0, b1)
            store(cb, b0, sb)

            @pl.when(i < _NCH // 2 - 1)
            def _():
                wait_store(cb, b0, sb)
                gather(cb + 2, (b0, b1), (gb0, gb1))

            return 0

        lax.fori_loop(0, _NCH // 2, body, 0)
        wait_store(_NCH - 2, a0, sa)
        wait_store(_NCH - 1, b0, sb)

    return k(down_out, d0.reshape(_IDX_ROWS, _CH), d1.reshape(_IDX_ROWS, _CH))


@jax.jit
def kernel(hidden_states, router_logits, gate_proj, up_proj, down_proj):
    w, d, bounds = _routing(router_logits)

    hs_sorted, w_sorted = _scatter_sc(hidden_states, d[:, 0] + 0, d[:, 1] + 0,
                                      w[:, 0] + 0, w[:, 1] + 0)

    down_out = _grouped_mlp(hs_sorted, gate_proj, up_proj, down_proj,
                            w_sorted, bounds)

    return _combine_sc(down_out, d[:, 0] + 0, d[:, 1] + 0)
